# speedup vs baseline: 1.6397x; 1.6397x over previous
import functools

import jax
import jax.numpy as jnp
from jax import lax
from jax.experimental import pallas as pl
from jax.experimental.pallas import tpu as pltpu
from jax.experimental.pallas import tpu_sc as plsc

D = 3584
N_IDS = 4096
NC, NS = 2, 16
NW = NC * NS
BPW = N_IDS // NW
CH = 16
NCHUNK = BPW // CH
NBUF = 2


@functools.partial(
    pl.kernel,
    out_type=jax.ShapeDtypeStruct((N_IDS, D), jnp.float32),
    mesh=plsc.VectorSubcoreMesh(core_axis_name="c", subcore_axis_name="s"),
    scratch_types=[
        pltpu.VMEM((BPW,), jnp.int32),
        pltpu.VMEM((NBUF, CH, D), jnp.float32),
        pltpu.SemaphoreType.DMA((NBUF,)),
        pltpu.SemaphoreType.DMA((NBUF,)),
    ],
)
def _gather_call(ids_hbm, table_hbm, out_hbm, idx_v, rows_v, in_sems, out_sems):
    wid = lax.axis_index("s") * NC + lax.axis_index("c")
    base = wid * BPW
    pltpu.sync_copy(ids_hbm.at[pl.ds(base, BPW)], idx_v)

    def gather(c, buf):
        pltpu.make_async_copy(
            table_hbm.at[idx_v.at[pl.ds(c * CH, CH)]],
            rows_v.at[buf],
            in_sems.at[buf],
        ).start()

    def wait_gather(c, buf):
        pltpu.make_async_copy(
            table_hbm.at[idx_v.at[pl.ds(c * CH, CH)]],
            rows_v.at[buf],
            in_sems.at[buf],
        ).wait()

    def put(c, buf):
        pltpu.make_async_copy(
            rows_v.at[buf],
            out_hbm.at[pl.ds(base + c * CH, CH)],
            out_sems.at[buf],
        ).start()

    def wait_put(c, buf):
        pltpu.make_async_copy(
            rows_v.at[buf],
            out_hbm.at[pl.ds(base + c * CH, CH)],
            out_sems.at[buf],
        ).wait()

    gather(0, 0)
    for c in range(NCHUNK):
        buf = c % NBUF
        nxt = (c + 1) % NBUF
        if c + 1 < NCHUNK:
            if c >= 1:
                wait_put(c - 1, nxt)
            gather(c + 1, nxt)
        wait_gather(c, buf)
        put(c, buf)
    for c in (NCHUNK - 2, NCHUNK - 1):
        wait_put(c, c % NBUF)


def kernel(input_ids, table):
    ids = input_ids.reshape(-1).astype(jnp.int32)
    out = _gather_call(ids, table)
    return out.reshape(input_ids.shape + (table.shape[1],))
